# baseline (speedup 1.0000x reference)
import functools

import jax
import jax.numpy as jnp
from jax import lax
from jax.experimental import pallas as pl
from jax.experimental.pallas import tpu as pltpu
from jax.experimental.pallas import tpu_sc as plsc

_BEV_EXTENT = (-32.0, -32.0, 32.0, 32.0)
_EPS = 1e-8
_Q = 512


def _main_body(a0_ref, a1_ref, pc0T_ref, pc1T_ref, tn0_ref, tn1_ref,
               fwt_ref, bwt_ref, out_ref, idx_ref):
    g = pl.program_id(0)
    nb = fwt_ref.shape[0]
    is_fw = g < nb

    src = jnp.where(is_fw, a0_ref[0], a1_ref[0])
    tT = jnp.where(is_fw, pc1T_ref[0], pc0T_ref[0])
    tn = jnp.where(is_fw, tn1_ref[0], tn0_ref[0])

    b = lax.rem(g, nb)
    trafo = jnp.where(is_fw, fwt_ref[b], bwt_ref[b])
    rot = trafo[:3, :3]
    trans = trafo[:3, 3]

    x0, y0, x1, y1 = _BEV_EXTENT
    n = src.shape[0]
    acc = [jnp.float32(0.0)] * 7

    for c in range(n // _Q):
        sl = slice(c * _Q, (c + 1) * _Q)
        blk = src[sl, :]
        p = blk[:, 0:3]
        fsrc = blk[:, 3:6]
        w = p + fsrc

        gram = lax.dot_general(w, tT, (((1,), (0,)), ((), ())),
                               preferred_element_type=jnp.float32)
        u = tn - 2.0 * gram
        umin = jnp.min(u, axis=1, keepdims=True)
        wn = jnp.sum(w * w, axis=1, keepdims=True)
        nn_d2 = jnp.maximum(wn + umin, 0.0)
        iota = lax.broadcasted_iota(jnp.int32, u.shape, 1)
        nn = jnp.min(jnp.where(u == umin, iota, n), axis=1,
                     keepdims=True)
        idx_ref[0, sl, :] = nn + g * n

        wx = w[:, 0:1]
        wy = w[:, 1:2]
        in_bev = ((wx >= x0) & (wx <= x1) & (wy >= y0) & (wy <= y1)
                  ).astype(jnp.float32)

        rp = lax.dot_general(p, rot, (((1,), (1,)), ((), ())),
                             preferred_element_type=jnp.float32)
        trafo_flow = rp + trans[None, :] - p
        serr = blk[:, 6:9] - trafo_flow
        serr2 = jnp.sum(serr * serr, axis=1, keepdims=True)
        sn = blk[:, 9:10]

        dis = blk[:, 10:11]
        valid = (dis == dis).astype(jnp.float32)

        acc[0] += jnp.sum(in_bev * nn_d2)
        acc[1] += jnp.sum(in_bev)
        acc[3] += jnp.sum(sn * serr2)
        acc[4] += jnp.sum(sn)
        acc[5] += jnp.sum(jnp.where(dis == dis, dis, 0.0))
        acc[6] += jnp.sum(valid)

    eye = jnp.eye(4, dtype=jnp.float32)
    sse = jnp.float32(0.0)
    for bb in range(nb):
        comp = jnp.dot(fwt_ref[bb], bwt_ref[bb],
                       preferred_element_type=jnp.float32)
        dlt = comp - eye
        sse = sse + jnp.sum(dlt * dlt)
    sse = sse * (g == 0).astype(jnp.float32)

    slots = lax.broadcasted_iota(jnp.int32, (1, 1, 8), 2)
    vals = acc + [sse]
    row = jnp.zeros((1, 1, 8), jnp.float32)
    for k, v in enumerate(vals):
        row = row + jnp.where(slots == k, v, 0.0)
    out_ref[...] = row


def _opp_body(a0_ref, a1_ref, fnn_ref, out_ref):
    g = pl.program_id(0)
    nb = pl.num_programs(0) // 2
    is_fw = g < nb

    src = jnp.where(is_fw, a0_ref[0], a1_ref[0])
    p = src[:, 0:3]
    fsrc = src[:, 3:6]
    w = p + fsrc
    x0, y0, x1, y1 = _BEV_EXTENT
    wx = w[:, 0:1]
    wy = w[:, 1:2]
    in_bev = ((wx >= x0) & (wx <= x1) & (wy >= y0) & (wy <= y1)
              ).astype(jnp.float32)
    opp = fsrc + fnn_ref[0][:, 0:3]
    opp_err = jnp.sum(opp * opp, axis=1, keepdims=True)
    val = jnp.sum(in_bev * opp_err)
    slots = lax.broadcasted_iota(jnp.int32, (1, 1, 8), 2)
    out_ref[...] = jnp.where(slots == 0, val, 0.0)


def _sc_gather(table, idx):
    gn, d = table.shape
    info = plsc.get_sparse_core_info()
    nw = info.num_cores * info.num_subcores
    b_per_w = gn // nw
    mesh = plsc.VectorSubcoreMesh(core_axis_name="c", subcore_axis_name="s")

    @functools.partial(
        pl.kernel, mesh=mesh,
        out_type=jax.ShapeDtypeStruct((gn, d), jnp.float32),
        scratch_types=[
            pltpu.VMEM((b_per_w,), jnp.int32),
            pltpu.VMEM((b_per_w, d), jnp.float32),
            pltpu.SemaphoreType.DMA,
        ],
    )
    def k(table_hbm, idx_hbm, out_hbm, idx_v, rows_v, sem):
        wid = lax.axis_index("s") * info.num_cores + lax.axis_index("c")
        base = wid * b_per_w
        pltpu.sync_copy(idx_hbm.at[pl.ds(base, b_per_w)], idx_v)
        pltpu.async_copy(table_hbm.at[idx_v], rows_v, sem).wait()
        pltpu.sync_copy(rows_v, out_hbm.at[pl.ds(base, b_per_w)])

    return k(table, idx)


def kernel(pc0, pc1, fw_aggregated_flow, bw_aggregated_flow, fw_static_flow,
           bw_static_flow, fw_static_aggr_trafo, bw_static_aggr_trafo,
           fw_staticness, bw_staticness, fw_disappearing, bw_disappearing):
    B, N, _ = pc0.shape
    G = 2 * B

    a0 = jnp.concatenate(
        [pc0, fw_aggregated_flow, fw_static_flow,
         fw_staticness[..., None], fw_disappearing[..., None]], axis=-1)
    a1 = jnp.concatenate(
        [pc1, bw_aggregated_flow, bw_static_flow,
         bw_staticness[..., None], bw_disappearing[..., None]], axis=-1)

    pc0T = pc0.transpose(0, 2, 1)
    pc1T = pc1.transpose(0, 2, 1)
    tn0 = jnp.sum(pc0 * pc0, axis=-1)[:, None, :]
    tn1 = jnp.sum(pc1 * pc1, axis=-1)[:, None, :]

    bspec = lambda shape: pl.BlockSpec(shape, lambda g: (lax.rem(g, B),)
                                       + (0,) * (len(shape) - 1))

    out, idx = pl.pallas_call(
        _main_body,
        grid=(G,),
        in_specs=[
            bspec((1, N, 11)),
            bspec((1, N, 11)),
            bspec((1, 3, N)),
            bspec((1, 3, N)),
            bspec((1, 1, N)),
            bspec((1, 1, N)),
            pl.BlockSpec((B, 4, 4), lambda g: (0, 0, 0)),
            pl.BlockSpec((B, 4, 4), lambda g: (0, 0, 0)),
        ],
        out_specs=[
            pl.BlockSpec((1, 1, 8), lambda g: (g, 0, 0)),
            pl.BlockSpec((1, N, 1), lambda g: (g, 0, 0)),
        ],
        out_shape=[
            jax.ShapeDtypeStruct((G, 1, 8), jnp.float32),
            jax.ShapeDtypeStruct((G, N, 1), jnp.int32),
        ],
        compiler_params=pltpu.CompilerParams(
            dimension_semantics=("arbitrary",)),
    )(a0, a1, pc0T, pc1T, tn0, tn1,
      fw_static_aggr_trafo, bw_static_aggr_trafo)

    ftgt = jnp.concatenate([bw_aggregated_flow, fw_aggregated_flow], axis=0)
    table = jnp.concatenate(
        [ftgt, jnp.zeros((G, N, 125), jnp.float32)], axis=-1).reshape(G * N, 128)
    fnn = _sc_gather(table, idx.reshape(G * N)).reshape(G, N, 128)

    opp = pl.pallas_call(
        _opp_body,
        grid=(G,),
        in_specs=[
            bspec((1, N, 11)),
            bspec((1, N, 11)),
            pl.BlockSpec((1, N, 128), lambda g: (g, 0, 0)),
        ],
        out_specs=pl.BlockSpec((1, 1, 8), lambda g: (g, 0, 0)),
        out_shape=jax.ShapeDtypeStruct((G, 1, 8), jnp.float32),
        compiler_params=pltpu.CompilerParams(
            dimension_semantics=("arbitrary",)),
    )(a0, a1, fnn)

    out = out.reshape(G, 8)
    opp = opp.reshape(G, 8)
    fw = out[:B]
    bw = out[B:]
    eps = jnp.float32(_EPS)

    def seg(rows, opp_rows):
        s = jnp.sum(rows, axis=0)
        den = s[1] + eps
        return s[0] / den, jnp.sum(opp_rows[:, 0]) / den, s[3] / (s[4] + eps)

    fw_fl, fw_opp, fw_static = seg(fw, opp[:B])
    bw_fl, bw_opp, bw_static = seg(bw, opp[B:])
    flow_loss = 0.5 * (fw_fl + bw_fl)
    opposite_flow_loss = 0.5 * (fw_opp + bw_opp)
    static_flow_loss = 0.5 * (fw_static + bw_static)
    occlusion_loss = jnp.sum(out[:, 5]) / (jnp.sum(out[:, 6]) + eps)
    trafo_loss = jnp.sum(out[:, 7]) / (B * 16.0)

    total = (static_flow_loss + trafo_loss + 0.1 * occlusion_loss
             + flow_loss + opposite_flow_loss)
    return total
